# Initial kernel scaffold; baseline (speedup 1.0000x reference)
import jax
import jax.numpy as jnp
from jax.experimental import pallas as pl


def kernel(raw_concept_ids, concept_ids, value_ids, position_ids, delta_ts, abs_ts, patient_contexts, raw_table, con_table, val_table, pos_table, rel_lin_w, rel_lin_b, rel_freq_w, rel_freq_b, abs_lin_w, abs_lin_b, abs_freq_w, abs_freq_b, time_proj_w, ctx_token, context_proj_w, final_proj_w, final_proj_b, ln_gamma, ln_beta):
    raise NotImplementedError("write your pallas kernel here")


# speedup vs baseline: 3.5300x; 3.5300x over previous
import functools
import math

import jax
import jax.numpy as jnp
from jax import lax
from jax.experimental import pallas as pl
from jax.experimental.pallas import tpu as pltpu
from jax.experimental.pallas import tpu_sc as plsc

D = 128
B = 1024
T = 200
N = B * T
NW = 32
CH = 128
PER_W = N // NW
NCHW = PER_W // CH
BB = 8
_INV_SQRT_D = 1.0 / math.sqrt(D)


def _sc_gather_body(t0, t1, t2, t3, i0, i1, i2, i3,
                    o0, o1, o2, o3, idx_v, rows_v,
                    g0, g1, g2, g3, s0, s1, s2, s3):
    tables = (t0, t1, t2, t3)
    outs = (o0, o1, o2, o3)
    gsems = (g0, g1, g2, g3)
    ssems = (s0, s1, s2, s3)
    wid = lax.axis_index("s") * 2 + lax.axis_index("c")
    for tab, iref in enumerate((i0, i1, i2, i3)):
        pltpu.sync_copy(iref.at[wid], idx_v.at[tab])

    def chunk(c, carry):
        hs = [pltpu.async_copy(tables[tab].at[idx_v.at[tab, c]],
                               rows_v.at[tab], gsems[tab])
              for tab in range(4)]
        row0 = wid * PER_W + c * CH
        ss = []
        for tab in range(4):
            hs[tab].wait()
            ss.append(pltpu.async_copy(rows_v.at[tab],
                                       outs[tab].at[pl.ds(row0, CH)],
                                       ssems[tab]))
        for tab in range(4):
            ss[tab].wait()
        return carry

    lax.fori_loop(0, NCHW, chunk, 0)


@functools.cache
def _sc_gather():
    return pl.kernel(
        _sc_gather_body,
        out_type=tuple(jax.ShapeDtypeStruct((N, D), jnp.float32)
                       for _ in range(4)),
        mesh=plsc.VectorSubcoreMesh(core_axis_name="c", subcore_axis_name="s"),
        scratch_types=[
            pltpu.VMEM((4, NCHW, CH), jnp.int32),
            pltpu.VMEM((4, CH, D), jnp.float32),
        ] + [pltpu.SemaphoreType.DMA] * 8,
    )


def _tc_body(frp, g0, g1, g2, g3, dts, ats, pc,
             w0, w1, w2, w3, mr, ma, ldv, lav, b2, cw, ct, gm, bt, out_ref):
    R = BB * T
    acc = jnp.dot(g0[...].reshape(R, D), w0[...],
                  preferred_element_type=jnp.float32)
    acc = acc + jnp.dot(g1[...].reshape(R, D), w1[...],
                        preferred_element_type=jnp.float32)
    acc = acc + jnp.dot(g2[...].reshape(R, D), w2[...],
                        preferred_element_type=jnp.float32)
    acc = acc + jnp.dot(g3[...].reshape(R, D), w3[...],
                        preferred_element_type=jnp.float32)

    td = dts[...]
    ta = ats[...]
    mrv = mr[...]
    mav = ma[...]
    tacc = td[..., None] * ldv[...].reshape(1, 1, D)
    tacc = tacc + ta[..., None] * lav[...].reshape(1, 1, D)
    for j in range(7):
        tacc = tacc + jnp.sin(td * frp[j] + frp[8 + j])[..., None] \
            * mrv[j].reshape(1, 1, D)
        tacc = tacc + jnp.sin(ta * frp[16 + j] + frp[24 + j])[..., None] \
            * mav[j].reshape(1, 1, D)

    ev = (acc.reshape(BB, T, D) + tacc + b2[...].reshape(1, 1, D)) * _INV_SQRT_D
    ctx = jnp.dot(pc[...], cw[...], preferred_element_type=jnp.float32) + ct[...]
    seq = jnp.concatenate([ctx[:, None, :], ev], axis=1)
    mean = jnp.mean(seq, axis=2, keepdims=True)
    xc = seq - mean
    var = jnp.mean(xc * xc, axis=2, keepdims=True)
    out_ref[...] = xc * lax.rsqrt(var + 1e-5) * gm[...].reshape(1, 1, D) \
        + bt[...].reshape(1, 1, D)


def _tc_finalize(frp, g0, g1, g2, g3, dts, ats, pc,
                 w0, w1, w2, w3, mr, ma, ldv, lav, b2, cw, ct, gm, bt,
                 interpret=False):
    full = lambda shape: pl.BlockSpec(shape, lambda i: (0,) * len(shape))
    return pl.pallas_call(
        _tc_body,
        grid=(B // BB,),
        in_specs=[
            pl.BlockSpec(memory_space=pltpu.SMEM),
        ] + [pl.BlockSpec((BB, T, D), lambda i: (i, 0, 0))] * 4 + [
            pl.BlockSpec((BB, T), lambda i: (i, 0)),
            pl.BlockSpec((BB, T), lambda i: (i, 0)),
            pl.BlockSpec((BB, 64), lambda i: (i, 0)),
        ] + [full((D, D))] * 4 + [
            full((8, D)), full((8, D)),
            full((1, D)), full((1, D)), full((1, D)),
            full((64, D)), full((1, D)), full((1, D)), full((1, D)),
        ],
        out_specs=pl.BlockSpec((BB, T + 1, D), lambda i: (i, 0, 0)),
        out_shape=jax.ShapeDtypeStruct((B, T + 1, D), jnp.float32),
        interpret=interpret,
    )(frp, g0, g1, g2, g3, dts, ats, pc,
      w0, w1, w2, w3, mr, ma, ldv, lav, b2, cw, ct, gm, bt)


def _prep_params(rel_lin_w, rel_lin_b, rel_freq_w, rel_freq_b,
                 abs_lin_w, abs_lin_b, abs_freq_w, abs_freq_b,
                 time_proj_w, ctx_token, context_proj_w,
                 final_proj_w, final_proj_b, ln_gamma, ln_beta):
    wt = final_proj_w[:, 4 * D:5 * D]
    m = time_proj_w.T @ wt.T
    ldv = (rel_lin_w[0, 0] * m[0]).reshape(1, D)
    lav = (abs_lin_w[0, 0] * m[8]).reshape(1, D)
    b2 = (final_proj_b + rel_lin_b[0] * m[0] + abs_lin_b[0] * m[8]).reshape(1, D)
    zrow = jnp.zeros((1, D), jnp.float32)
    mr = jnp.concatenate([m[1:8], zrow], axis=0)
    ma = jnp.concatenate([m[9:16], zrow], axis=0)
    frp = jnp.zeros((32,), jnp.float32)
    frp = frp.at[0:7].set(rel_freq_w[:, 0]).at[8:15].set(rel_freq_b)
    frp = frp.at[16:23].set(abs_freq_w[:, 0]).at[24:31].set(abs_freq_b)
    ws = [final_proj_w[:, k * D:(k + 1) * D].T for k in range(4)]
    cw = context_proj_w.T
    ct = ctx_token.reshape(1, D)
    gm = ln_gamma.reshape(1, D)
    bt = ln_beta.reshape(1, D)
    return frp, ws, mr, ma, ldv, lav, b2, cw, ct, gm, bt


def kernel(raw_concept_ids, concept_ids, value_ids, position_ids, delta_ts,
           abs_ts, patient_contexts, raw_table, con_table, val_table,
           pos_table, rel_lin_w, rel_lin_b, rel_freq_w, rel_freq_b,
           abs_lin_w, abs_lin_b, abs_freq_w, abs_freq_b, time_proj_w,
           ctx_token, context_proj_w, final_proj_w, final_proj_b,
           ln_gamma, ln_beta):
    ids = [a.astype(jnp.int32).reshape(NW, NCHW, CH)
           for a in (raw_concept_ids, concept_ids, value_ids, position_ids)]
    g0, g1, g2, g3 = _sc_gather()(raw_table, con_table, val_table, pos_table,
                                  *ids)
    frp, ws, mr, ma, ldv, lav, b2, cw, ct, gm, bt = _prep_params(
        rel_lin_w, rel_lin_b, rel_freq_w, rel_freq_b,
        abs_lin_w, abs_lin_b, abs_freq_w, abs_freq_b,
        time_proj_w, ctx_token, context_proj_w,
        final_proj_w, final_proj_b, ln_gamma, ln_beta)
    gs = [g.reshape(B, T, D) for g in (g0, g1, g2, g3)]
    return _tc_finalize(frp, *gs, delta_ts, abs_ts, patient_contexts,
                        *ws, mr, ma, ldv, lav, b2, cw, ct, gm, bt)


# speedup vs baseline: 5.3025x; 1.5021x over previous
import functools
import math

import jax
import jax.numpy as jnp
from jax import lax
from jax.experimental import pallas as pl
from jax.experimental.pallas import tpu as pltpu
from jax.experimental.pallas import tpu_sc as plsc

D = 128
B = 1024
T = 200
N = B * T
NW = 32
CH = 128
PER_W = N // NW
NCHW = PER_W // CH
BB = 8
_INV_SQRT_D = 1.0 / math.sqrt(D)


def _sc_gather_body(t0, t1, t2, t3, i0, i1, i2, i3,
                    o0, o1, o2, o3, idx_v, rows_v,
                    g0, g1, g2, g3, s0, s1, s2, s3):
    tables = (t0, t1, t2, t3)
    outs = (o0, o1, o2, o3)
    gsems = (g0, g1, g2, g3)
    ssems = (s0, s1, s2, s3)
    wid = lax.axis_index("s") * 2 + lax.axis_index("c")
    for tab, iref in enumerate((i0, i1, i2, i3)):
        pltpu.sync_copy(iref.at[wid], idx_v.at[tab])

    def chunk(c, carry):
        hs = [pltpu.async_copy(tables[tab].at[idx_v.at[tab, c]],
                               rows_v.at[tab], gsems[tab])
              for tab in range(4)]
        row0 = wid * PER_W + c * CH
        ss = []
        for tab in range(4):
            hs[tab].wait()
            ss.append(pltpu.async_copy(rows_v.at[tab],
                                       outs[tab].at[pl.ds(row0, CH)],
                                       ssems[tab]))
        for tab in range(4):
            ss[tab].wait()
        return carry

    lax.fori_loop(0, NCHW, chunk, 0)


@functools.cache
def _sc_gather():
    return pl.kernel(
        _sc_gather_body,
        out_type=tuple(jax.ShapeDtypeStruct((N, D), jnp.float32)
                       for _ in range(4)),
        mesh=plsc.VectorSubcoreMesh(core_axis_name="c", subcore_axis_name="s"),
        scratch_types=[
            pltpu.VMEM((4, NCHW, CH), jnp.int32),
            pltpu.VMEM((4, CH, D), jnp.float32),
        ] + [pltpu.SemaphoreType.DMA] * 8,
    )


def _ln(x, gm, bt):
    mean = jnp.mean(x, axis=-1, keepdims=True)
    xc = x - mean
    var = jnp.mean(xc * xc, axis=-1, keepdims=True)
    return xc * lax.rsqrt(var + 1e-5) * gm + bt


def _tc_body(frp, g0, g1, g2, g3, dts, ats, pc,
             w0, w1, w2, w3, m2, b2, cw, ct, gm, bt, out_ref):
    acc = jnp.dot(g0[...], w0[...], preferred_element_type=jnp.float32)
    acc = acc + jnp.dot(g1[...], w1[...], preferred_element_type=jnp.float32)
    acc = acc + jnp.dot(g2[...], w2[...], preferred_element_type=jnp.float32)
    acc = acc + jnp.dot(g3[...], w3[...], preferred_element_type=jnp.float32)

    R = BB * T
    tdl = dts[...].reshape(R)
    tal = ats[...].reshape(R)
    feats = [tdl, tal]
    for j in range(7):
        feats.append(jnp.sin(tdl * frp[j] + frp[8 + j]))
    for j in range(7):
        feats.append(jnp.sin(tal * frp[16 + j] + frp[24 + j]))
    s = jnp.stack(feats, axis=0)
    c = lax.dot_general(s, m2[...], (((0,), (0,)), ((), ())),
                        preferred_element_type=jnp.float32)

    ev = (acc + c + b2[...]) * _INV_SQRT_D
    y = _ln(ev, gm[...], bt[...])
    ctx = jnp.dot(pc[...], cw[...], preferred_element_type=jnp.float32) + ct[...]
    cy = _ln(ctx, gm[...], bt[...])
    out_ref[:, 0:1, :] = cy[:, None, :]
    out_ref[:, 1:, :] = y.reshape(BB, T, D)


def _tc_finalize(frp, g0, g1, g2, g3, dts, ats, pc,
                 w0, w1, w2, w3, m2, b2, cw, ct, gm, bt):
    R = BB * T
    full = lambda shape: pl.BlockSpec(shape, lambda i: (0,) * len(shape))
    return pl.pallas_call(
        _tc_body,
        grid=(B // BB,),
        in_specs=[
            pl.BlockSpec(memory_space=pltpu.SMEM),
        ] + [pl.BlockSpec((R, D), lambda i: (i, 0))] * 4 + [
            pl.BlockSpec((1, 1, R), lambda i: (i, 0, 0)),
            pl.BlockSpec((1, 1, R), lambda i: (i, 0, 0)),
            pl.BlockSpec((BB, 64), lambda i: (i, 0)),
        ] + [full((D, D))] * 4 + [
            full((16, D)), full((1, D)),
            full((64, D)), full((1, D)), full((1, D)), full((1, D)),
        ],
        out_specs=pl.BlockSpec((BB, T + 1, D), lambda i: (i, 0, 0)),
        out_shape=jax.ShapeDtypeStruct((B, T + 1, D), jnp.float32),
    )(frp, g0, g1, g2, g3, dts, ats, pc,
      w0, w1, w2, w3, m2, b2, cw, ct, gm, bt)


def _prep_params(rel_lin_w, rel_lin_b, rel_freq_w, rel_freq_b,
                 abs_lin_w, abs_lin_b, abs_freq_w, abs_freq_b,
                 time_proj_w, ctx_token, context_proj_w,
                 final_proj_w, final_proj_b, ln_gamma, ln_beta):
    wt = final_proj_w[:, 4 * D:5 * D]
    m = time_proj_w.T @ wt.T
    b2 = (final_proj_b + rel_lin_b[0] * m[0] + abs_lin_b[0] * m[8]).reshape(1, D)
    m2 = jnp.concatenate([
        (rel_lin_w[0, 0] * m[0]).reshape(1, D),
        (abs_lin_w[0, 0] * m[8]).reshape(1, D),
        m[1:8], m[9:16]], axis=0)
    frp = jnp.zeros((32,), jnp.float32)
    frp = frp.at[0:7].set(rel_freq_w[:, 0]).at[8:15].set(rel_freq_b)
    frp = frp.at[16:23].set(abs_freq_w[:, 0]).at[24:31].set(abs_freq_b)
    ws = [final_proj_w[:, k * D:(k + 1) * D].T for k in range(4)]
    cw = context_proj_w.T
    ct = ctx_token.reshape(1, D)
    gm = ln_gamma.reshape(1, D)
    bt = ln_beta.reshape(1, D)
    return frp, ws, m2, b2, cw, ct, gm, bt


def kernel(raw_concept_ids, concept_ids, value_ids, position_ids, delta_ts,
           abs_ts, patient_contexts, raw_table, con_table, val_table,
           pos_table, rel_lin_w, rel_lin_b, rel_freq_w, rel_freq_b,
           abs_lin_w, abs_lin_b, abs_freq_w, abs_freq_b, time_proj_w,
           ctx_token, context_proj_w, final_proj_w, final_proj_b,
           ln_gamma, ln_beta):
    ids = [a.astype(jnp.int32).reshape(NW, NCHW, CH)
           for a in (raw_concept_ids, concept_ids, value_ids, position_ids)]
    g0, g1, g2, g3 = _sc_gather()(raw_table, con_table, val_table, pos_table,
                                  *ids)
    frp, ws, m2, b2, cw, ct, gm, bt = _prep_params(
        rel_lin_w, rel_lin_b, rel_freq_w, rel_freq_b,
        abs_lin_w, abs_lin_b, abs_freq_w, abs_freq_b,
        time_proj_w, ctx_token, context_proj_w,
        final_proj_w, final_proj_b, ln_gamma, ln_beta)
    rsh = (B // BB, 1, BB * T)
    return _tc_finalize(frp, g0, g1, g2, g3, delta_ts.reshape(rsh),
                        abs_ts.reshape(rsh), patient_contexts,
                        *ws, m2, b2, cw, ct, gm, bt)


# speedup vs baseline: 5.3353x; 1.0062x over previous
import functools
import math

import jax
import jax.numpy as jnp
from jax import lax
from jax.experimental import pallas as pl
from jax.experimental.pallas import tpu as pltpu
from jax.experimental.pallas import tpu_sc as plsc

D = 128
B = 1024
T = 200
N = B * T
NW = 32
CH = 64
PER_W = N // NW
NCHW = PER_W // CH
BB = 8
_INV_SQRT_D = 1.0 / math.sqrt(D)


def _sc_gather_body(t0, t1, t2, t3, i0, i1, i2, i3,
                    o0, o1, o2, o3, idx_v, rows_v,
                    g0, g1, g2, g3, s0, s1, s2, s3):
    tables = (t0, t1, t2, t3)
    outs = (o0, o1, o2, o3)
    gsems = (g0, g1, g2, g3)
    ssems = (s0, s1, s2, s3)
    wid = lax.axis_index("s") * 2 + lax.axis_index("c")
    base = wid * PER_W
    for tab, iref in enumerate((i0, i1, i2, i3)):
        pltpu.sync_copy(iref.at[wid], idx_v.at[tab])

    def gathers(c, b):
        return [pltpu.async_copy(tables[tab].at[idx_v.at[tab, c]],
                                 rows_v.at[tab, b], gsems[tab])
                for tab in range(4)]

    def issue_stores(c, b):
        row0 = base + c * CH
        for tab in range(4):
            pltpu.async_copy(rows_v.at[tab, b],
                             outs[tab].at[pl.ds(row0, CH)], ssems[tab])

    def wait_stores(b):
        for tab in range(4):
            pltpu.make_async_copy(rows_v.at[tab, b],
                                  outs[tab].at[pl.ds(0, CH)],
                                  ssems[tab]).wait()

    def step(s, carry):
        c0 = s * 2

        @pl.when(s > 0)
        def _():
            wait_stores(0)
        hg0 = gathers(c0, 0)

        @pl.when(s > 0)
        def _():
            wait_stores(1)
        hg1 = gathers(c0 + 1, 1)
        for h in hg0:
            h.wait()
        issue_stores(c0, 0)
        for h in hg1:
            h.wait()
        issue_stores(c0 + 1, 1)
        return carry

    lax.fori_loop(0, NCHW // 2, step, 0)
    wait_stores(0)
    wait_stores(1)


@functools.cache
def _sc_gather():
    return pl.kernel(
        _sc_gather_body,
        out_type=tuple(jax.ShapeDtypeStruct((N, D), jnp.float32)
                       for _ in range(4)),
        mesh=plsc.VectorSubcoreMesh(core_axis_name="c", subcore_axis_name="s"),
        scratch_types=[
            pltpu.VMEM((4, NCHW, CH), jnp.int32),
            pltpu.VMEM((4, 2, CH, D), jnp.float32),
        ] + [pltpu.SemaphoreType.DMA] * 8,
    )


def _ln(x, gm, bt):
    mean = jnp.mean(x, axis=-1, keepdims=True)
    xc = x - mean
    var = jnp.mean(xc * xc, axis=-1, keepdims=True)
    return xc * lax.rsqrt(var + 1e-5) * gm + bt


def _tc_body(frp, g0, g1, g2, g3, dts, ats, pc,
             w0, w1, w2, w3, m2, b2, cw, ct, gm, bt, out_ref):
    R = BB * T
    acc = jnp.dot(g0[...], w0[...], preferred_element_type=jnp.float32)
    acc = acc + jnp.dot(g1[...], w1[...], preferred_element_type=jnp.float32)
    acc = acc + jnp.dot(g2[...], w2[...], preferred_element_type=jnp.float32)
    acc = acc + jnp.dot(g3[...], w3[...], preferred_element_type=jnp.float32)

    tdl = dts[...].reshape(R)
    tal = ats[...].reshape(R)
    feats = [tdl, tal]
    for j in range(7):
        feats.append(jnp.sin(tdl * frp[j] + frp[8 + j]))
    for j in range(7):
        feats.append(jnp.sin(tal * frp[16 + j] + frp[24 + j]))
    s = jnp.stack(feats, axis=0)
    c = lax.dot_general(s, m2[...], (((0,), (0,)), ((), ())),
                        preferred_element_type=jnp.float32)

    ev = (acc + c + b2[...]) * _INV_SQRT_D
    y = _ln(ev, gm[...], bt[...])
    ctx = jnp.dot(pc[...], cw[...], preferred_element_type=jnp.float32) + ct[...]
    cy = _ln(ctx, gm[...], bt[...])
    out_ref[:, 0:1, :] = cy[:, None, :]
    out_ref[:, 1:, :] = y.reshape(BB, T, D)


def _tc_finalize(frp, g0, g1, g2, g3, dts, ats, pc,
                 w0, w1, w2, w3, m2, b2, cw, ct, gm, bt):
    R = BB * T
    full = lambda shape: pl.BlockSpec(shape, lambda i: (0,) * len(shape))
    return pl.pallas_call(
        _tc_body,
        grid=(B // BB,),
        in_specs=[
            pl.BlockSpec(memory_space=pltpu.SMEM),
        ] + [pl.BlockSpec((R, D), lambda i: (i, 0))] * 4 + [
            pl.BlockSpec((1, 1, R), lambda i: (i, 0, 0)),
            pl.BlockSpec((1, 1, R), lambda i: (i, 0, 0)),
            pl.BlockSpec((BB, 64), lambda i: (i, 0)),
        ] + [full((D, D))] * 4 + [
            full((16, D)), full((1, D)),
            full((64, D)), full((1, D)), full((1, D)), full((1, D)),
        ],
        out_specs=pl.BlockSpec((BB, T + 1, D), lambda i: (i, 0, 0)),
        out_shape=jax.ShapeDtypeStruct((B, T + 1, D), jnp.float32),
    )(frp, g0, g1, g2, g3, dts, ats, pc,
      w0, w1, w2, w3, m2, b2, cw, ct, gm, bt)


def _prep_params(rel_lin_w, rel_lin_b, rel_freq_w, rel_freq_b,
                 abs_lin_w, abs_lin_b, abs_freq_w, abs_freq_b,
                 time_proj_w, ctx_token, context_proj_w,
                 final_proj_w, final_proj_b, ln_gamma, ln_beta):
    wt = final_proj_w[:, 4 * D:5 * D]
    m = time_proj_w.T @ wt.T
    b2 = (final_proj_b + rel_lin_b[0] * m[0] + abs_lin_b[0] * m[8]).reshape(1, D)
    m2 = jnp.concatenate([
        (rel_lin_w[0, 0] * m[0]).reshape(1, D),
        (abs_lin_w[0, 0] * m[8]).reshape(1, D),
        m[1:8], m[9:16]], axis=0)
    frp = jnp.zeros((32,), jnp.float32)
    frp = frp.at[0:7].set(rel_freq_w[:, 0]).at[8:15].set(rel_freq_b)
    frp = frp.at[16:23].set(abs_freq_w[:, 0]).at[24:31].set(abs_freq_b)
    ws = [final_proj_w[:, k * D:(k + 1) * D].T for k in range(4)]
    cw = context_proj_w.T
    ct = ctx_token.reshape(1, D)
    gm = ln_gamma.reshape(1, D)
    bt = ln_beta.reshape(1, D)
    return frp, ws, m2, b2, cw, ct, gm, bt


def kernel(raw_concept_ids, concept_ids, value_ids, position_ids, delta_ts,
           abs_ts, patient_contexts, raw_table, con_table, val_table,
           pos_table, rel_lin_w, rel_lin_b, rel_freq_w, rel_freq_b,
           abs_lin_w, abs_lin_b, abs_freq_w, abs_freq_b, time_proj_w,
           ctx_token, context_proj_w, final_proj_w, final_proj_b,
           ln_gamma, ln_beta):
    ids = [a.astype(jnp.int32).reshape(NW, NCHW, CH)
           for a in (raw_concept_ids, concept_ids, value_ids, position_ids)]
    g0, g1, g2, g3 = _sc_gather()(raw_table, con_table, val_table, pos_table,
                                  *ids)
    frp, ws, m2, b2, cw, ct, gm, bt = _prep_params(
        rel_lin_w, rel_lin_b, rel_freq_w, rel_freq_b,
        abs_lin_w, abs_lin_b, abs_freq_w, abs_freq_b,
        time_proj_w, ctx_token, context_proj_w,
        final_proj_w, final_proj_b, ln_gamma, ln_beta)
    rsh = (B // BB, 1, BB * T)
    return _tc_finalize(frp, g0, g1, g2, g3, delta_ts.reshape(rsh),
                        abs_ts.reshape(rsh), patient_contexts,
                        *ws, m2, b2, cw, ct, gm, bt)


# speedup vs baseline: 5.5396x; 1.0383x over previous
import functools
import math

import jax
import jax.numpy as jnp
from jax import lax
from jax.experimental import pallas as pl
from jax.experimental.pallas import tpu as pltpu
from jax.experimental.pallas import tpu_sc as plsc

D = 128
B = 1024
T = 200
N = B * T
NW = 32
CH = 64
PER_W = N // NW
NCHW = PER_W // CH
BB = 8
_INV_SQRT_D = 1.0 / math.sqrt(D)


def _sc_gather_body(t0, t1, t2, t3, i0, i1, i2, i3,
                    o0, o1, o2, o3, idx_v, rows_v,
                    g0, g1, g2, g3, s0, s1, s2, s3):
    tables = (t0, t1, t2, t3)
    outs = (o0, o1, o2, o3)
    gsems = (g0, g1, g2, g3)
    ssems = (s0, s1, s2, s3)
    wid = lax.axis_index("s") * 2 + lax.axis_index("c")
    base = wid * PER_W
    for tab, iref in enumerate((i0, i1, i2, i3)):
        pltpu.sync_copy(iref.at[wid], idx_v.at[tab])

    def gathers(c, b):
        return [pltpu.async_copy(tables[tab].at[idx_v.at[tab, c]],
                                 rows_v.at[tab, b], gsems[tab])
                for tab in range(4)]

    def issue_stores(c, b):
        row0 = base + c * CH
        for tab in range(4):
            pltpu.async_copy(rows_v.at[tab, b],
                             outs[tab].at[pl.ds(row0, CH)], ssems[tab])

    def wait_stores(b):
        for tab in range(4):
            pltpu.make_async_copy(rows_v.at[tab, b],
                                  outs[tab].at[pl.ds(0, CH)],
                                  ssems[tab]).wait()

    def step(s, carry):
        c0 = s * 2

        @pl.when(s > 0)
        def _():
            wait_stores(0)
        hg0 = gathers(c0, 0)

        @pl.when(s > 0)
        def _():
            wait_stores(1)
        hg1 = gathers(c0 + 1, 1)
        for h in hg0:
            h.wait()
        issue_stores(c0, 0)
        for h in hg1:
            h.wait()
        issue_stores(c0 + 1, 1)
        return carry

    lax.fori_loop(0, NCHW // 2, step, 0)
    wait_stores(0)
    wait_stores(1)


@functools.cache
def _sc_gather():
    return pl.kernel(
        _sc_gather_body,
        out_type=tuple(jax.ShapeDtypeStruct((N, D), jnp.float32)
                       for _ in range(4)),
        mesh=plsc.VectorSubcoreMesh(core_axis_name="c", subcore_axis_name="s"),
        scratch_types=[
            pltpu.VMEM((4, NCHW, CH), jnp.int32),
            pltpu.VMEM((4, 2, CH, D), jnp.float32),
        ] + [pltpu.SemaphoreType.DMA] * 8,
    )


def _ln(x, gm, bt):
    mean = jnp.mean(x, axis=-1, keepdims=True)
    xc = x - mean
    var = jnp.mean(xc * xc, axis=-1, keepdims=True)
    return xc * lax.rsqrt(var + 1e-5) * gm + bt


def _tc_body(frp, g0, g1, g2, g3, dts, ats, pc,
             w0, w1, w2, w3, m2, b2, cw, ct, gm, bt, out_ref):
    i = pl.program_id(0)

    @pl.when(i == 0)
    def _():
        ctx = jnp.dot(pc[...], cw[...],
                      preferred_element_type=jnp.float32) + ct[...]
        out_ref[...] = _ln(ctx, gm[...], bt[...])

    @pl.when(i > 0)
    def _():
        acc = jnp.dot(g0[...], w0[...], preferred_element_type=jnp.float32)
        acc = acc + jnp.dot(g1[...], w1[...],
                            preferred_element_type=jnp.float32)
        acc = acc + jnp.dot(g2[...], w2[...],
                            preferred_element_type=jnp.float32)
        acc = acc + jnp.dot(g3[...], w3[...],
                            preferred_element_type=jnp.float32)

        tdl = dts[...].reshape(B)
        tal = ats[...].reshape(B)
        feats = [tdl, tal]
        for j in range(7):
            feats.append(jnp.sin(tdl * frp[j] + frp[8 + j]))
        for j in range(7):
            feats.append(jnp.sin(tal * frp[16 + j] + frp[24 + j]))
        s = jnp.stack(feats, axis=0)
        c = lax.dot_general(s, m2[...], (((0,), (0,)), ((), ())),
                            preferred_element_type=jnp.float32)

        ev = (acc + c + b2[...]) * _INV_SQRT_D
        out_ref[...] = _ln(ev, gm[...], bt[...])


def _tc_finalize(frp, g0, g1, g2, g3, dts, ats, pc,
                 w0, w1, w2, w3, m2, b2, cw, ct, gm, bt):
    full = lambda shape: pl.BlockSpec(shape, lambda i: (0,) * len(shape))
    gmap = lambda i: (jnp.maximum(i - 1, 0), 0)
    tmap = lambda i: (jnp.maximum(i - 1, 0), 0, 0)
    return pl.pallas_call(
        _tc_body,
        grid=(T + 1,),
        in_specs=[
            pl.BlockSpec(memory_space=pltpu.SMEM),
        ] + [pl.BlockSpec((B, D), gmap)] * 4 + [
            pl.BlockSpec((1, 1, B), tmap),
            pl.BlockSpec((1, 1, B), tmap),
            full((B, 64)),
        ] + [full((D, D))] * 4 + [
            full((16, D)), full((1, D)),
            full((64, D)), full((1, D)), full((1, D)), full((1, D)),
        ],
        out_specs=pl.BlockSpec((B, D), lambda i: (i, 0)),
        out_shape=jax.ShapeDtypeStruct(((T + 1) * B, D), jnp.float32),
    )(frp, g0, g1, g2, g3, dts, ats, pc,
      w0, w1, w2, w3, m2, b2, cw, ct, gm, bt)


def _prep_params(rel_lin_w, rel_lin_b, rel_freq_w, rel_freq_b,
                 abs_lin_w, abs_lin_b, abs_freq_w, abs_freq_b,
                 time_proj_w, ctx_token, context_proj_w,
                 final_proj_w, final_proj_b, ln_gamma, ln_beta):
    wt = final_proj_w[:, 4 * D:5 * D]
    m = time_proj_w.T @ wt.T
    b2 = (final_proj_b + rel_lin_b[0] * m[0] + abs_lin_b[0] * m[8]).reshape(1, D)
    m2 = jnp.concatenate([
        (rel_lin_w[0, 0] * m[0]).reshape(1, D),
        (abs_lin_w[0, 0] * m[8]).reshape(1, D),
        m[1:8], m[9:16]], axis=0)
    frp = jnp.zeros((32,), jnp.float32)
    frp = frp.at[0:7].set(rel_freq_w[:, 0]).at[8:15].set(rel_freq_b)
    frp = frp.at[16:23].set(abs_freq_w[:, 0]).at[24:31].set(abs_freq_b)
    ws = [final_proj_w[:, k * D:(k + 1) * D].T for k in range(4)]
    cw = context_proj_w.T
    ct = ctx_token.reshape(1, D)
    gm = ln_gamma.reshape(1, D)
    bt = ln_beta.reshape(1, D)
    return frp, ws, m2, b2, cw, ct, gm, bt


def kernel(raw_concept_ids, concept_ids, value_ids, position_ids, delta_ts,
           abs_ts, patient_contexts, raw_table, con_table, val_table,
           pos_table, rel_lin_w, rel_lin_b, rel_freq_w, rel_freq_b,
           abs_lin_w, abs_lin_b, abs_freq_w, abs_freq_b, time_proj_w,
           ctx_token, context_proj_w, final_proj_w, final_proj_b,
           ln_gamma, ln_beta):
    ids = [a.astype(jnp.int32).T.reshape(NW, NCHW, CH)
           for a in (raw_concept_ids, concept_ids, value_ids, position_ids)]
    g0, g1, g2, g3 = _sc_gather()(raw_table, con_table, val_table, pos_table,
                                  *ids)
    frp, ws, m2, b2, cw, ct, gm, bt = _prep_params(
        rel_lin_w, rel_lin_b, rel_freq_w, rel_freq_b,
        abs_lin_w, abs_lin_b, abs_freq_w, abs_freq_b,
        time_proj_w, ctx_token, context_proj_w,
        final_proj_w, final_proj_b, ln_gamma, ln_beta)
    out = _tc_finalize(frp, g0, g1, g2, g3, delta_ts.T.reshape(T, 1, B),
                       abs_ts.T.reshape(T, 1, B), patient_contexts,
                       *ws, m2, b2, cw, ct, gm, bt)
    return out.reshape(T + 1, B, D).transpose(1, 0, 2)


# speedup vs baseline: 6.1246x; 1.1056x over previous
import functools
import math

import jax
import jax.numpy as jnp
from jax import lax
from jax.experimental import pallas as pl
from jax.experimental.pallas import tpu as pltpu
from jax.experimental.pallas import tpu_sc as plsc

D = 128
B = 1024
T = 200
N = B * T
NW = 32
CH = 64
HALVES = 2
NH = N // HALVES
TH = T // HALVES
_INV_SQRT_D = 1.0 / math.sqrt(D)


@functools.cache
def _sc_gather(n_rows):
    per_w = n_rows // NW
    nchw = per_w // CH

    def body(t0, t1, t2, t3, i0, i1, i2, i3,
             o0, o1, o2, o3, idx_v, rows_v,
             g0, g1, g2, g3, s0, s1, s2, s3):
        tables = (t0, t1, t2, t3)
        outs = (o0, o1, o2, o3)
        gsems = (g0, g1, g2, g3)
        ssems = (s0, s1, s2, s3)
        wid = lax.axis_index("s") * 2 + lax.axis_index("c")
        base = wid * per_w
        for tab, iref in enumerate((i0, i1, i2, i3)):
            pltpu.sync_copy(iref.at[wid], idx_v.at[tab])

        def gathers(c, b):
            return [pltpu.async_copy(tables[tab].at[idx_v.at[tab, c]],
                                     rows_v.at[tab, b], gsems[tab])
                    for tab in range(4)]

        def issue_stores(c, b):
            row0 = base + c * CH
            for tab in range(4):
                pltpu.async_copy(rows_v.at[tab, b],
                                 outs[tab].at[pl.ds(row0, CH)], ssems[tab])

        def wait_stores(b):
            for tab in range(4):
                pltpu.make_async_copy(rows_v.at[tab, b],
                                      outs[tab].at[pl.ds(0, CH)],
                                      ssems[tab]).wait()

        def step(s, carry):
            c0 = s * 2

            @pl.when(s > 0)
            def _():
                wait_stores(0)
            hg0 = gathers(c0, 0)

            @pl.when(s > 0)
            def _():
                wait_stores(1)
            hg1 = gathers(c0 + 1, 1)
            for h in hg0:
                h.wait()
            issue_stores(c0, 0)
            for h in hg1:
                h.wait()
            issue_stores(c0 + 1, 1)
            return carry

        lax.fori_loop(0, nchw // 2, step, 0)
        wait_stores(0)
        wait_stores(1)

    return pl.kernel(
        body,
        out_type=tuple(jax.ShapeDtypeStruct((n_rows, D), jnp.float32)
                       for _ in range(4)),
        mesh=plsc.VectorSubcoreMesh(core_axis_name="c", subcore_axis_name="s"),
        scratch_types=[
            pltpu.VMEM((4, nchw, CH), jnp.int32),
            pltpu.VMEM((4, 2, CH, D), jnp.float32),
        ] + [pltpu.SemaphoreType.DMA] * 8,
    )


def _ln(x, gm, bt):
    mean = jnp.mean(x, axis=-1, keepdims=True)
    xc = x - mean
    var = jnp.mean(xc * xc, axis=-1, keepdims=True)
    return xc * lax.rsqrt(var + 1e-5) * gm + bt


def _ev_block(frp, g0, g1, g2, g3, dts, ats, w0, w1, w2, w3, m2, b2):
    acc = jnp.dot(g0[...], w0[...], preferred_element_type=jnp.float32)
    acc = acc + jnp.dot(g1[...], w1[...], preferred_element_type=jnp.float32)
    acc = acc + jnp.dot(g2[...], w2[...], preferred_element_type=jnp.float32)
    acc = acc + jnp.dot(g3[...], w3[...], preferred_element_type=jnp.float32)

    tdl = dts[...].reshape(B)
    tal = ats[...].reshape(B)
    feats = [tdl, tal]
    for j in range(7):
        feats.append(jnp.sin(tdl * frp[j] + frp[8 + j]))
    for j in range(7):
        feats.append(jnp.sin(tal * frp[16 + j] + frp[24 + j]))
    s = jnp.stack(feats, axis=0)
    c = lax.dot_general(s, m2[...], (((0,), (0,)), ((), ())),
                        preferred_element_type=jnp.float32)
    return (acc + c + b2[...]) * _INV_SQRT_D


def _tc_body_ctx(frp, g0, g1, g2, g3, dts, ats, pc,
                 w0, w1, w2, w3, m2, b2, cw, ct, gm, bt, out_ref):
    i = pl.program_id(0)

    @pl.when(i == 0)
    def _():
        ctx = jnp.dot(pc[...], cw[...],
                      preferred_element_type=jnp.float32) + ct[...]
        out_ref[...] = _ln(ctx, gm[...], bt[...])

    @pl.when(i > 0)
    def _():
        ev = _ev_block(frp, g0, g1, g2, g3, dts, ats,
                       w0, w1, w2, w3, m2, b2)
        out_ref[...] = _ln(ev, gm[...], bt[...])


def _tc_body_ev(frp, g0, g1, g2, g3, dts, ats,
                w0, w1, w2, w3, m2, b2, gm, bt, prev, out_ref):
    ev = _ev_block(frp, g0, g1, g2, g3, dts, ats, w0, w1, w2, w3, m2, b2)
    out_ref[...] = _ln(ev, gm[...], bt[...])


_FULL = lambda shape: pl.BlockSpec(shape, lambda i: (0,) * len(shape))
_WSPECS = [_FULL((D, D))] * 4 + [_FULL((16, D)), _FULL((1, D))]


def _tc_half1(frp, g0, g1, g2, g3, dts, ats, pc,
              w0, w1, w2, w3, m2, b2, cw, ct, gm, bt):
    gmap = lambda i: (jnp.maximum(i - 1, 0), 0)
    tmap = lambda i: (jnp.maximum(i - 1, 0), 0, 0)
    return pl.pallas_call(
        _tc_body_ctx,
        grid=(TH + 1,),
        in_specs=[
            pl.BlockSpec(memory_space=pltpu.SMEM),
        ] + [pl.BlockSpec((B, D), gmap)] * 4 + [
            pl.BlockSpec((1, 1, B), tmap),
            pl.BlockSpec((1, 1, B), tmap),
            _FULL((B, 64)),
        ] + _WSPECS + [
            _FULL((64, D)), _FULL((1, D)), _FULL((1, D)), _FULL((1, D)),
        ],
        out_specs=pl.BlockSpec((B, D), lambda i: (i, 0)),
        out_shape=jax.ShapeDtypeStruct(((T + 1) * B, D), jnp.float32),
    )(frp, g0, g1, g2, g3, dts, ats, pc,
      w0, w1, w2, w3, m2, b2, cw, ct, gm, bt)


def _tc_half2(frp, g0, g1, g2, g3, dts, ats,
              w0, w1, w2, w3, m2, b2, gm, bt, prev):
    gmap = lambda i: (i, 0)
    tmap = lambda i: (i, 0, 0)
    return pl.pallas_call(
        _tc_body_ev,
        grid=(TH,),
        in_specs=[
            pl.BlockSpec(memory_space=pltpu.SMEM),
        ] + [pl.BlockSpec((B, D), gmap)] * 4 + [
            pl.BlockSpec((1, 1, B), tmap),
            pl.BlockSpec((1, 1, B), tmap),
        ] + _WSPECS + [
            _FULL((1, D)), _FULL((1, D)),
            pl.BlockSpec(memory_space=pl.ANY),
        ],
        out_specs=pl.BlockSpec((B, D), lambda i: (i + TH + 1, 0)),
        out_shape=jax.ShapeDtypeStruct(((T + 1) * B, D), jnp.float32),
        input_output_aliases={15: 0},
    )(frp, g0, g1, g2, g3, dts, ats,
      w0, w1, w2, w3, m2, b2, gm, bt, prev)


def _prep_params(rel_lin_w, rel_lin_b, rel_freq_w, rel_freq_b,
                 abs_lin_w, abs_lin_b, abs_freq_w, abs_freq_b,
                 time_proj_w, ctx_token, context_proj_w,
                 final_proj_w, final_proj_b, ln_gamma, ln_beta):
    wt = final_proj_w[:, 4 * D:5 * D]
    m = time_proj_w.T @ wt.T
    b2 = (final_proj_b + rel_lin_b[0] * m[0] + abs_lin_b[0] * m[8]).reshape(1, D)
    m2 = jnp.concatenate([
        (rel_lin_w[0, 0] * m[0]).reshape(1, D),
        (abs_lin_w[0, 0] * m[8]).reshape(1, D),
        m[1:8], m[9:16]], axis=0)
    frp = jnp.zeros((32,), jnp.float32)
    frp = frp.at[0:7].set(rel_freq_w[:, 0]).at[8:15].set(rel_freq_b)
    frp = frp.at[16:23].set(abs_freq_w[:, 0]).at[24:31].set(abs_freq_b)
    ws = [final_proj_w[:, k * D:(k + 1) * D].T for k in range(4)]
    cw = context_proj_w.T
    ct = ctx_token.reshape(1, D)
    gm = ln_gamma.reshape(1, D)
    bt = ln_beta.reshape(1, D)
    return frp, ws, m2, b2, cw, ct, gm, bt


def kernel(raw_concept_ids, concept_ids, value_ids, position_ids, delta_ts,
           abs_ts, patient_contexts, raw_table, con_table, val_table,
           pos_table, rel_lin_w, rel_lin_b, rel_freq_w, rel_freq_b,
           abs_lin_w, abs_lin_b, abs_freq_w, abs_freq_b, time_proj_w,
           ctx_token, context_proj_w, final_proj_w, final_proj_b,
           ln_gamma, ln_beta):
    nchw_h = NH // NW // CH
    ids = [a.astype(jnp.int32).T.reshape(HALVES, NW, nchw_h, CH)
           for a in (raw_concept_ids, concept_ids, value_ids, position_ids)]
    tabs = (raw_table, con_table, val_table, pos_table)
    gh0 = _sc_gather(NH)(*tabs, *(x[0] for x in ids))
    gh1 = _sc_gather(NH)(*tabs, *(x[1] for x in ids))
    frp, ws, m2, b2, cw, ct, gm, bt = _prep_params(
        rel_lin_w, rel_lin_b, rel_freq_w, rel_freq_b,
        abs_lin_w, abs_lin_b, abs_freq_w, abs_freq_b,
        time_proj_w, ctx_token, context_proj_w,
        final_proj_w, final_proj_b, ln_gamma, ln_beta)
    dts = delta_ts.T.reshape(HALVES, TH, 1, B)
    ats = abs_ts.T.reshape(HALVES, TH, 1, B)
    out = _tc_half1(frp, *gh0, dts[0], ats[0], patient_contexts,
                    *ws, m2, b2, cw, ct, gm, bt)
    out = _tc_half2(frp, *gh1, dts[1], ats[1], *ws, m2, b2, gm, bt, out)
    return out.reshape(T + 1, B, D).transpose(1, 0, 2)


# speedup vs baseline: 7.3630x; 1.2022x over previous
import functools
import math

import jax
import jax.numpy as jnp
from jax import lax
from jax.experimental import pallas as pl
from jax.experimental.pallas import tpu as pltpu
from jax.experimental.pallas import tpu_sc as plsc

D = 128
B = 1024
T = 200
N = B * T
NW = 32
CH = 64
HALVES = 2
NH = N // HALVES
TH = T // HALVES
_INV_SQRT_D = 1.0 / math.sqrt(D)


@functools.cache
def _sc_gather(n_rows):
    per_w = n_rows // NW
    nchw = per_w // CH

    def body(t0, t1, t2, t3, i0, i1, i2, i3,
             o0, o1, idx_v, rows_v, pk_v,
             g0, g1, g2, g3, s0, s1):
        tables = (t0, t1, t2, t3)
        outs = (o0, o1)
        gsems = (g0, g1, g2, g3)
        ssems = (s0, s1)
        wid = lax.axis_index("s") * 2 + lax.axis_index("c")
        base = wid * per_w
        for tab, iref in enumerate((i0, i1, i2, i3)):
            pltpu.sync_copy(iref.at[wid], idx_v.at[tab])

        def gathers(c, b):
            return [pltpu.async_copy(tables[tab].at[idx_v.at[tab, c]],
                                     rows_v.at[tab, b], gsems[tab])
                    for tab in range(4)]

        def pack_chunk(b):
            def prow(r, carry):
                for pair in range(2):
                    for g in range(8):
                        a = rows_v[2 * pair, b, r, pl.ds(16 * g, 16)]
                        z = rows_v[2 * pair + 1, b, r, pl.ds(16 * g, 16)]
                        ai = lax.bitcast_convert_type(a, jnp.int32)
                        zi = lax.bitcast_convert_type(z, jnp.int32)
                        ar = lax.shift_right_logical(ai + 0x8000, 16)
                        zr = (zi + 0x8000) & jnp.int32(-65536)
                        pk_v[pair, b, r, pl.ds(16 * g, 16)] = ar | zr
                return carry
            lax.fori_loop(0, CH, prow, 0)

        def issue_stores(c, b):
            row0 = base + c * CH
            for pair in range(2):
                pltpu.async_copy(pk_v.at[pair, b],
                                 outs[pair].at[pl.ds(row0, CH)], ssems[pair])

        def wait_stores(b):
            for pair in range(2):
                pltpu.make_async_copy(pk_v.at[pair, b],
                                      outs[pair].at[pl.ds(0, CH)],
                                      ssems[pair]).wait()

        def step(s, carry):
            c0 = s * 2

            @pl.when(s > 0)
            def _():
                wait_stores(0)
            hg0 = gathers(c0, 0)

            @pl.when(s > 0)
            def _():
                wait_stores(1)
            hg1 = gathers(c0 + 1, 1)
            for h in hg0:
                h.wait()
            pack_chunk(0)
            issue_stores(c0, 0)
            for h in hg1:
                h.wait()
            pack_chunk(1)
            issue_stores(c0 + 1, 1)
            return carry

        lax.fori_loop(0, nchw // 2, step, 0)
        wait_stores(0)
        wait_stores(1)

    return pl.kernel(
        body,
        out_type=tuple(jax.ShapeDtypeStruct((n_rows, D), jnp.int32)
                       for _ in range(2)),
        mesh=plsc.VectorSubcoreMesh(core_axis_name="c", subcore_axis_name="s"),
        scratch_types=[
            pltpu.VMEM((4, nchw, CH), jnp.int32),
            pltpu.VMEM((4, 2, CH, D), jnp.float32),
            pltpu.VMEM((2, 2, CH, D), jnp.int32),
        ] + [pltpu.SemaphoreType.DMA] * 6,
    )


def _ln(x, gm, bt):
    mean = jnp.mean(x, axis=-1, keepdims=True)
    xc = x - mean
    var = jnp.mean(xc * xc, axis=-1, keepdims=True)
    return xc * lax.rsqrt(var + 1e-5) * gm + bt


def _ev_block(frp, g01, g23, dts, ats, w0, w1, w2, w3, m2, b2):
    acc = None
    for g, wl, wh in ((g01, w0, w1), (g23, w2, w3)):
        x = g[...]
        lo = lax.bitcast_convert_type(x << 16, jnp.float32).astype(jnp.bfloat16)
        hi = lax.bitcast_convert_type(x & jnp.int32(-65536),
                                      jnp.float32).astype(jnp.bfloat16)
        p = jnp.dot(lo, wl[...], preferred_element_type=jnp.float32) \
            + jnp.dot(hi, wh[...], preferred_element_type=jnp.float32)
        acc = p if acc is None else acc + p

    tdl = dts[...].reshape(B)
    tal = ats[...].reshape(B)
    feats = [tdl, tal]
    for j in range(7):
        feats.append(jnp.sin(tdl * frp[j] + frp[8 + j]))
    for j in range(7):
        feats.append(jnp.sin(tal * frp[16 + j] + frp[24 + j]))
    s = jnp.stack(feats, axis=0)
    c = lax.dot_general(s, m2[...], (((0,), (0,)), ((), ())),
                        preferred_element_type=jnp.float32)
    return (acc + c + b2[...]) * _INV_SQRT_D


def _tc_body_ctx(frp, g01, g23, dts, ats, pc,
                 w0, w1, w2, w3, m2, b2, cw, ct, gm, bt, out_ref):
    i = pl.program_id(0)

    @pl.when(i == 0)
    def _():
        ctx = jnp.dot(pc[...], cw[...],
                      preferred_element_type=jnp.float32) + ct[...]
        out_ref[...] = _ln(ctx, gm[...], bt[...])

    @pl.when(i > 0)
    def _():
        ev = _ev_block(frp, g01, g23, dts, ats, w0, w1, w2, w3, m2, b2)
        out_ref[...] = _ln(ev, gm[...], bt[...])


def _tc_body_ev(frp, g01, g23, dts, ats,
                w0, w1, w2, w3, m2, b2, gm, bt, prev, out_ref):
    ev = _ev_block(frp, g01, g23, dts, ats, w0, w1, w2, w3, m2, b2)
    out_ref[...] = _ln(ev, gm[...], bt[...])


_FULL = lambda shape: pl.BlockSpec(shape, lambda i: (0,) * len(shape))
_WSPECS = [_FULL((D, D))] * 4 + [_FULL((16, D)), _FULL((1, D))]


def _tc_half1(frp, g01, g23, dts, ats, pc,
              w0, w1, w2, w3, m2, b2, cw, ct, gm, bt):
    gmap = lambda i: (jnp.maximum(i - 1, 0), 0)
    tmap = lambda i: (jnp.maximum(i - 1, 0), 0, 0)
    return pl.pallas_call(
        _tc_body_ctx,
        grid=(TH + 1,),
        in_specs=[
            pl.BlockSpec(memory_space=pltpu.SMEM),
        ] + [pl.BlockSpec((B, D), gmap)] * 2 + [
            pl.BlockSpec((1, 1, B), tmap),
            pl.BlockSpec((1, 1, B), tmap),
            _FULL((B, 64)),
        ] + _WSPECS + [
            _FULL((64, D)), _FULL((1, D)), _FULL((1, D)), _FULL((1, D)),
        ],
        out_specs=pl.BlockSpec((B, D), lambda i: (i, 0)),
        out_shape=jax.ShapeDtypeStruct(((T + 1) * B, D), jnp.float32),
    )(frp, g01, g23, dts, ats, pc,
      w0, w1, w2, w3, m2, b2, cw, ct, gm, bt)


def _tc_half2(frp, g01, g23, dts, ats,
              w0, w1, w2, w3, m2, b2, gm, bt, prev):
    gmap = lambda i: (i, 0)
    tmap = lambda i: (i, 0, 0)
    return pl.pallas_call(
        _tc_body_ev,
        grid=(TH,),
        in_specs=[
            pl.BlockSpec(memory_space=pltpu.SMEM),
        ] + [pl.BlockSpec((B, D), gmap)] * 2 + [
            pl.BlockSpec((1, 1, B), tmap),
            pl.BlockSpec((1, 1, B), tmap),
        ] + _WSPECS + [
            _FULL((1, D)), _FULL((1, D)),
            pl.BlockSpec(memory_space=pl.ANY),
        ],
        out_specs=pl.BlockSpec((B, D), lambda i: (i + TH + 1, 0)),
        out_shape=jax.ShapeDtypeStruct(((T + 1) * B, D), jnp.float32),
        input_output_aliases={13: 0},
    )(frp, g01, g23, dts, ats,
      w0, w1, w2, w3, m2, b2, gm, bt, prev)


def _prep_params(rel_lin_w, rel_lin_b, rel_freq_w, rel_freq_b,
                 abs_lin_w, abs_lin_b, abs_freq_w, abs_freq_b,
                 time_proj_w, ctx_token, context_proj_w,
                 final_proj_w, final_proj_b, ln_gamma, ln_beta):
    wt = final_proj_w[:, 4 * D:5 * D]
    m = time_proj_w.T @ wt.T
    b2 = (final_proj_b + rel_lin_b[0] * m[0] + abs_lin_b[0] * m[8]).reshape(1, D)
    m2 = jnp.concatenate([
        (rel_lin_w[0, 0] * m[0]).reshape(1, D),
        (abs_lin_w[0, 0] * m[8]).reshape(1, D),
        m[1:8], m[9:16]], axis=0)
    frp = jnp.zeros((32,), jnp.float32)
    frp = frp.at[0:7].set(rel_freq_w[:, 0]).at[8:15].set(rel_freq_b)
    frp = frp.at[16:23].set(abs_freq_w[:, 0]).at[24:31].set(abs_freq_b)
    ws = [final_proj_w[:, k * D:(k + 1) * D].T.astype(jnp.bfloat16)
          for k in range(4)]
    cw = context_proj_w.T
    ct = ctx_token.reshape(1, D)
    gm = ln_gamma.reshape(1, D)
    bt = ln_beta.reshape(1, D)
    return frp, ws, m2, b2, cw, ct, gm, bt


def kernel(raw_concept_ids, concept_ids, value_ids, position_ids, delta_ts,
           abs_ts, patient_contexts, raw_table, con_table, val_table,
           pos_table, rel_lin_w, rel_lin_b, rel_freq_w, rel_freq_b,
           abs_lin_w, abs_lin_b, abs_freq_w, abs_freq_b, time_proj_w,
           ctx_token, context_proj_w, final_proj_w, final_proj_b,
           ln_gamma, ln_beta):
    nchw_h = NH // NW // CH
    ids = [a.astype(jnp.int32).T.reshape(HALVES, NW, nchw_h, CH)
           for a in (raw_concept_ids, concept_ids, value_ids, position_ids)]
    tabs = (raw_table, con_table, val_table, pos_table)
    gh0 = _sc_gather(NH)(*tabs, *(x[0] for x in ids))
    gh1 = _sc_gather(NH)(*tabs, *(x[1] for x in ids))
    frp, ws, m2, b2, cw, ct, gm, bt = _prep_params(
        rel_lin_w, rel_lin_b, rel_freq_w, rel_freq_b,
        abs_lin_w, abs_lin_b, abs_freq_w, abs_freq_b,
        time_proj_w, ctx_token, context_proj_w,
        final_proj_w, final_proj_b, ln_gamma, ln_beta)
    dts = delta_ts.T.reshape(HALVES, TH, 1, B)
    ats = abs_ts.T.reshape(HALVES, TH, 1, B)
    out = _tc_half1(frp, gh0[0], gh0[1], dts[0], ats[0], patient_contexts,
                    *ws, m2, b2, cw, ct, gm, bt)
    out = _tc_half2(frp, gh1[0], gh1[1], dts[1], ats[1],
                    *ws, m2, b2, gm, bt, out)
    return out.reshape(T + 1, B, D).transpose(1, 0, 2)


# speedup vs baseline: 7.7017x; 1.0460x over previous
import functools
import math

import jax
import jax.numpy as jnp
from jax import lax
from jax.experimental import pallas as pl
from jax.experimental.pallas import tpu as pltpu
from jax.experimental.pallas import tpu_sc as plsc

D = 128
B = 1024
T = 200
N = B * T
NW = 32
CH = 32
HALVES = 4
NH = N // HALVES
TH = T // HALVES
_INV_SQRT_D = 1.0 / math.sqrt(D)


@functools.cache
def _sc_gather(n_rows):
    per_w = n_rows // NW
    nchw = per_w // CH

    def body(t0, t1, t2, t3, i0, i1, i2, i3,
             o0, o1, idx_v, rows_v, pk_v,
             g0, g1, g2, g3, s0, s1):
        tables = (t0, t1, t2, t3)
        outs = (o0, o1)
        gsems = (g0, g1, g2, g3)
        ssems = (s0, s1)
        wid = lax.axis_index("s") * 2 + lax.axis_index("c")
        base = wid * per_w
        for tab, iref in enumerate((i0, i1, i2, i3)):
            pltpu.sync_copy(iref.at[wid], idx_v.at[tab])

        def gathers(c, b):
            return [pltpu.async_copy(tables[tab].at[idx_v.at[tab, c]],
                                     rows_v.at[tab, b], gsems[tab])
                    for tab in range(4)]

        def pack_chunk(b):
            def prow(r, carry):
                for pair in range(2):
                    for g in range(8):
                        a = rows_v[2 * pair, b, r, pl.ds(16 * g, 16)]
                        z = rows_v[2 * pair + 1, b, r, pl.ds(16 * g, 16)]
                        ai = lax.bitcast_convert_type(a, jnp.int32)
                        zi = lax.bitcast_convert_type(z, jnp.int32)
                        ar = lax.shift_right_logical(ai + 0x8000, 16)
                        zr = (zi + 0x8000) & jnp.int32(-65536)
                        pk_v[pair, b, r, pl.ds(16 * g, 16)] = ar | zr
                return carry
            lax.fori_loop(0, CH, prow, 0)

        def issue_stores(c, b):
            row0 = base + c * CH
            for pair in range(2):
                pltpu.async_copy(pk_v.at[pair, b],
                                 outs[pair].at[pl.ds(row0, CH)], ssems[pair])

        def wait_stores(b):
            for pair in range(2):
                pltpu.make_async_copy(pk_v.at[pair, b],
                                      outs[pair].at[pl.ds(0, CH)],
                                      ssems[pair]).wait()

        def step(s, carry):
            c0 = s * 2

            @pl.when(s > 0)
            def _():
                wait_stores(0)
            hg0 = gathers(c0, 0)

            @pl.when(s > 0)
            def _():
                wait_stores(1)
            hg1 = gathers(c0 + 1, 1)
            for h in hg0:
                h.wait()
            pack_chunk(0)
            issue_stores(c0, 0)
            for h in hg1:
                h.wait()
            pack_chunk(1)
            issue_stores(c0 + 1, 1)
            return carry

        lax.fori_loop(0, nchw // 2, step, 0)
        wait_stores(0)
        wait_stores(1)

    return pl.kernel(
        body,
        out_type=tuple(jax.ShapeDtypeStruct((n_rows, D), jnp.int32)
                       for _ in range(2)),
        mesh=plsc.VectorSubcoreMesh(core_axis_name="c", subcore_axis_name="s"),
        scratch_types=[
            pltpu.VMEM((4, nchw, CH), jnp.int32),
            pltpu.VMEM((4, 2, CH, D), jnp.float32),
            pltpu.VMEM((2, 2, CH, D), jnp.int32),
        ] + [pltpu.SemaphoreType.DMA] * 6,
    )


def _ln(x, gm, bt):
    mean = jnp.mean(x, axis=-1, keepdims=True)
    xc = x - mean
    var = jnp.mean(xc * xc, axis=-1, keepdims=True)
    return xc * lax.rsqrt(var + 1e-5) * gm + bt


def _ev_block(frp, g01, g23, dts, ats, w0, w1, w2, w3, m2, b2):
    acc = None
    for g, wl, wh in ((g01, w0, w1), (g23, w2, w3)):
        x = g[...]
        lo = lax.bitcast_convert_type(x << 16, jnp.float32).astype(jnp.bfloat16)
        hi = lax.bitcast_convert_type(x & jnp.int32(-65536),
                                      jnp.float32).astype(jnp.bfloat16)
        p = jnp.dot(lo, wl[...], preferred_element_type=jnp.float32) \
            + jnp.dot(hi, wh[...], preferred_element_type=jnp.float32)
        acc = p if acc is None else acc + p

    tdl = dts[...].reshape(B)
    tal = ats[...].reshape(B)
    feats = [tdl, tal]
    for j in range(7):
        feats.append(jnp.sin(tdl * frp[j] + frp[8 + j]))
    for j in range(7):
        feats.append(jnp.sin(tal * frp[16 + j] + frp[24 + j]))
    s = jnp.stack(feats, axis=0)
    c = lax.dot_general(s, m2[...], (((0,), (0,)), ((), ())),
                        preferred_element_type=jnp.float32)
    return (acc + c + b2[...]) * _INV_SQRT_D


def _tc_body_ctx(frp, g01, g23, dts, ats, pc,
                 w0, w1, w2, w3, m2, b2, cw, ct, gm, bt, out_ref):
    i = pl.program_id(0)

    @pl.when(i == 0)
    def _():
        ctx = jnp.dot(pc[...], cw[...],
                      preferred_element_type=jnp.float32) + ct[...]
        out_ref[...] = _ln(ctx, gm[...], bt[...])

    @pl.when(i > 0)
    def _():
        ev = _ev_block(frp, g01, g23, dts, ats, w0, w1, w2, w3, m2, b2)
        out_ref[...] = _ln(ev, gm[...], bt[...])


def _tc_body_ev(frp, g01, g23, dts, ats,
                w0, w1, w2, w3, m2, b2, gm, bt, prev, out_ref):
    ev = _ev_block(frp, g01, g23, dts, ats, w0, w1, w2, w3, m2, b2)
    out_ref[...] = _ln(ev, gm[...], bt[...])


_FULL = lambda shape: pl.BlockSpec(shape, lambda i: (0,) * len(shape))
_WSPECS = [_FULL((D, D))] * 4 + [_FULL((16, D)), _FULL((1, D))]


def _tc_half1(frp, g01, g23, dts, ats, pc,
              w0, w1, w2, w3, m2, b2, cw, ct, gm, bt):
    gmap = lambda i: (jnp.maximum(i - 1, 0), 0)
    tmap = lambda i: (jnp.maximum(i - 1, 0), 0, 0)
    return pl.pallas_call(
        _tc_body_ctx,
        grid=(TH + 1,),
        in_specs=[
            pl.BlockSpec(memory_space=pltpu.SMEM),
        ] + [pl.BlockSpec((B, D), gmap)] * 2 + [
            pl.BlockSpec((1, 1, B), tmap),
            pl.BlockSpec((1, 1, B), tmap),
            _FULL((B, 64)),
        ] + _WSPECS + [
            _FULL((64, D)), _FULL((1, D)), _FULL((1, D)), _FULL((1, D)),
        ],
        out_specs=pl.BlockSpec((B, D), lambda i: (i, 0)),
        out_shape=jax.ShapeDtypeStruct(((T + 1) * B, D), jnp.float32),
    )(frp, g01, g23, dts, ats, pc,
      w0, w1, w2, w3, m2, b2, cw, ct, gm, bt)


def _tc_half2(frp, g01, g23, dts, ats,
              w0, w1, w2, w3, m2, b2, gm, bt, prev, off):
    gmap = lambda i: (i, 0)
    tmap = lambda i: (i, 0, 0)
    return pl.pallas_call(
        _tc_body_ev,
        grid=(TH,),
        in_specs=[
            pl.BlockSpec(memory_space=pltpu.SMEM),
        ] + [pl.BlockSpec((B, D), gmap)] * 2 + [
            pl.BlockSpec((1, 1, B), tmap),
            pl.BlockSpec((1, 1, B), tmap),
        ] + _WSPECS + [
            _FULL((1, D)), _FULL((1, D)),
            pl.BlockSpec(memory_space=pl.ANY),
        ],
        out_specs=pl.BlockSpec((B, D), lambda i: (i + off, 0)),
        out_shape=jax.ShapeDtypeStruct(((T + 1) * B, D), jnp.float32),
        input_output_aliases={13: 0},
    )(frp, g01, g23, dts, ats,
      w0, w1, w2, w3, m2, b2, gm, bt, prev)


def _prep_params(rel_lin_w, rel_lin_b, rel_freq_w, rel_freq_b,
                 abs_lin_w, abs_lin_b, abs_freq_w, abs_freq_b,
                 time_proj_w, ctx_token, context_proj_w,
                 final_proj_w, final_proj_b, ln_gamma, ln_beta):
    wt = final_proj_w[:, 4 * D:5 * D]
    m = time_proj_w.T @ wt.T
    b2 = (final_proj_b + rel_lin_b[0] * m[0] + abs_lin_b[0] * m[8]).reshape(1, D)
    m2 = jnp.concatenate([
        (rel_lin_w[0, 0] * m[0]).reshape(1, D),
        (abs_lin_w[0, 0] * m[8]).reshape(1, D),
        m[1:8], m[9:16]], axis=0)
    frp = jnp.zeros((32,), jnp.float32)
    frp = frp.at[0:7].set(rel_freq_w[:, 0]).at[8:15].set(rel_freq_b)
    frp = frp.at[16:23].set(abs_freq_w[:, 0]).at[24:31].set(abs_freq_b)
    ws = [final_proj_w[:, k * D:(k + 1) * D].T.astype(jnp.bfloat16)
          for k in range(4)]
    cw = context_proj_w.T
    ct = ctx_token.reshape(1, D)
    gm = ln_gamma.reshape(1, D)
    bt = ln_beta.reshape(1, D)
    return frp, ws, m2, b2, cw, ct, gm, bt


def kernel(raw_concept_ids, concept_ids, value_ids, position_ids, delta_ts,
           abs_ts, patient_contexts, raw_table, con_table, val_table,
           pos_table, rel_lin_w, rel_lin_b, rel_freq_w, rel_freq_b,
           abs_lin_w, abs_lin_b, abs_freq_w, abs_freq_b, time_proj_w,
           ctx_token, context_proj_w, final_proj_w, final_proj_b,
           ln_gamma, ln_beta):
    nchw_h = NH // NW // CH
    ids = [a.astype(jnp.int32).T.reshape(HALVES, NW, nchw_h, CH)
           for a in (raw_concept_ids, concept_ids, value_ids, position_ids)]
    tabs = (raw_table, con_table, val_table, pos_table)
    gh = [_sc_gather(NH)(*tabs, *(x[h] for x in ids)) for h in range(HALVES)]
    frp, ws, m2, b2, cw, ct, gm, bt = _prep_params(
        rel_lin_w, rel_lin_b, rel_freq_w, rel_freq_b,
        abs_lin_w, abs_lin_b, abs_freq_w, abs_freq_b,
        time_proj_w, ctx_token, context_proj_w,
        final_proj_w, final_proj_b, ln_gamma, ln_beta)
    dts = delta_ts.T.reshape(HALVES, TH, 1, B)
    ats = abs_ts.T.reshape(HALVES, TH, 1, B)
    out = _tc_half1(frp, gh[0][0], gh[0][1], dts[0], ats[0], patient_contexts,
                    *ws, m2, b2, cw, ct, gm, bt)
    for h in range(1, HALVES):
        out = _tc_half2(frp, gh[h][0], gh[h][1], dts[h], ats[h],
                        *ws, m2, b2, gm, bt, out, 1 + h * TH)
    return out.reshape(T + 1, B, D).transpose(1, 0, 2)


# speedup vs baseline: 8.1076x; 1.0527x over previous
import functools
import math

import jax
import jax.numpy as jnp
from jax import lax
from jax.experimental import pallas as pl
from jax.experimental.pallas import tpu as pltpu
from jax.experimental.pallas import tpu_sc as plsc

D = 128
B = 1024
T = 200
N = B * T
NW = 32
CH = 64
HALVES = 5
NH = N // HALVES
TH = T // HALVES
_INV_SQRT_D = 1.0 / math.sqrt(D)


@functools.cache
def _sc_gather(n_rows):
    per_w = n_rows // NW
    nchw = per_w // CH

    def body(t0, t1, t2, t3, i0, i1, i2, i3,
             o0, o1, idx_v, rows_v, pk_v,
             g0, g1, g2, g3, s0, s1):
        tables = (t0, t1, t2, t3)
        outs = (o0, o1)
        gsems = (g0, g1, g2, g3)
        ssems = (s0, s1)
        wid = lax.axis_index("s") * 2 + lax.axis_index("c")
        base = wid * per_w
        for tab, iref in enumerate((i0, i1, i2, i3)):
            pltpu.sync_copy(iref.at[wid], idx_v.at[tab])

        def gathers(c, b):
            return [pltpu.async_copy(tables[tab].at[idx_v.at[tab, c]],
                                     rows_v.at[tab, b], gsems[tab])
                    for tab in range(4)]

        def pack_chunk(b):
            def prow(r, carry):
                for pair in range(2):
                    for g in range(8):
                        a = rows_v[2 * pair, b, r, pl.ds(16 * g, 16)]
                        z = rows_v[2 * pair + 1, b, r, pl.ds(16 * g, 16)]
                        ai = lax.bitcast_convert_type(a, jnp.int32)
                        zi = lax.bitcast_convert_type(z, jnp.int32)
                        ar = lax.shift_right_logical(ai + 0x8000, 16)
                        zr = (zi + 0x8000) & jnp.int32(-65536)
                        pk_v[pair, b, r, pl.ds(16 * g, 16)] = ar | zr
                return carry
            lax.fori_loop(0, CH, prow, 0)

        def issue_stores(c, b):
            row0 = base + c * CH
            for pair in range(2):
                pltpu.async_copy(pk_v.at[pair, b],
                                 outs[pair].at[pl.ds(row0, CH)], ssems[pair])

        def wait_stores(b):
            for pair in range(2):
                pltpu.make_async_copy(pk_v.at[pair, b],
                                      outs[pair].at[pl.ds(0, CH)],
                                      ssems[pair]).wait()

        def step(s, carry):
            c0 = s * 2

            @pl.when(s > 0)
            def _():
                wait_stores(0)
            hg0 = gathers(c0, 0)

            @pl.when(s > 0)
            def _():
                wait_stores(1)
            hg1 = gathers(c0 + 1, 1)
            for h in hg0:
                h.wait()
            pack_chunk(0)
            issue_stores(c0, 0)
            for h in hg1:
                h.wait()
            pack_chunk(1)
            issue_stores(c0 + 1, 1)
            return carry

        lax.fori_loop(0, nchw // 2, step, 0)
        wait_stores(0)
        wait_stores(1)

    return pl.kernel(
        body,
        out_type=tuple(jax.ShapeDtypeStruct((n_rows, D), jnp.int32)
                       for _ in range(2)),
        mesh=plsc.VectorSubcoreMesh(core_axis_name="c", subcore_axis_name="s"),
        scratch_types=[
            pltpu.VMEM((4, nchw, CH), jnp.int32),
            pltpu.VMEM((4, 2, CH, D), jnp.float32),
            pltpu.VMEM((2, 2, CH, D), jnp.int32),
        ] + [pltpu.SemaphoreType.DMA] * 6,
    )


def _ln(x, gm, bt):
    mean = jnp.mean(x, axis=-1, keepdims=True)
    xc = x - mean
    var = jnp.mean(xc * xc, axis=-1, keepdims=True)
    return xc * lax.rsqrt(var + 1e-5) * gm + bt


def _ev_block(frp, g01, g23, dts, ats, w0, w1, w2, w3, m2, b2):
    acc = None
    for g, wl, wh in ((g01, w0, w1), (g23, w2, w3)):
        x = g[...]
        lo = lax.bitcast_convert_type(x << 16, jnp.float32).astype(jnp.bfloat16)
        hi = lax.bitcast_convert_type(x & jnp.int32(-65536),
                                      jnp.float32).astype(jnp.bfloat16)
        p = jnp.dot(lo, wl[...], preferred_element_type=jnp.float32) \
            + jnp.dot(hi, wh[...], preferred_element_type=jnp.float32)
        acc = p if acc is None else acc + p

    tdl = dts[...].reshape(B)
    tal = ats[...].reshape(B)
    feats = [tdl, tal]
    for j in range(7):
        feats.append(jnp.sin(tdl * frp[j] + frp[8 + j]))
    for j in range(7):
        feats.append(jnp.sin(tal * frp[16 + j] + frp[24 + j]))
    s = jnp.stack(feats, axis=0)
    c = lax.dot_general(s, m2[...], (((0,), (0,)), ((), ())),
                        preferred_element_type=jnp.float32)
    return (acc + c + b2[...]) * _INV_SQRT_D


def _tc_body_ctx(frp, g01, g23, dts, ats, pc,
                 w0, w1, w2, w3, m2, b2, cw, ct, gm, bt, out_ref):
    i = pl.program_id(0)

    @pl.when(i == 0)
    def _():
        ctx = jnp.dot(pc[...], cw[...],
                      preferred_element_type=jnp.float32) + ct[...]
        out_ref[...] = _ln(ctx, gm[...], bt[...])

    @pl.when(i > 0)
    def _():
        ev = _ev_block(frp, g01, g23, dts, ats, w0, w1, w2, w3, m2, b2)
        out_ref[...] = _ln(ev, gm[...], bt[...])


def _tc_body_ev(frp, g01, g23, dts, ats,
                w0, w1, w2, w3, m2, b2, gm, bt, prev, out_ref):
    ev = _ev_block(frp, g01, g23, dts, ats, w0, w1, w2, w3, m2, b2)
    out_ref[...] = _ln(ev, gm[...], bt[...])


_FULL = lambda shape: pl.BlockSpec(shape, lambda i: (0,) * len(shape))
_WSPECS = [_FULL((D, D))] * 4 + [_FULL((16, D)), _FULL((1, D))]


def _tc_half1(frp, g01, g23, dts, ats, pc,
              w0, w1, w2, w3, m2, b2, cw, ct, gm, bt):
    gmap = lambda i: (jnp.maximum(i - 1, 0), 0)
    tmap = lambda i: (jnp.maximum(i - 1, 0), 0, 0)
    return pl.pallas_call(
        _tc_body_ctx,
        grid=(TH + 1,),
        in_specs=[
            pl.BlockSpec(memory_space=pltpu.SMEM),
        ] + [pl.BlockSpec((B, D), gmap)] * 2 + [
            pl.BlockSpec((1, 1, B), tmap),
            pl.BlockSpec((1, 1, B), tmap),
            _FULL((B, 64)),
        ] + _WSPECS + [
            _FULL((64, D)), _FULL((1, D)), _FULL((1, D)), _FULL((1, D)),
        ],
        out_specs=pl.BlockSpec((B, D), lambda i: (i, 0)),
        out_shape=jax.ShapeDtypeStruct(((T + 1) * B, D), jnp.float32),
    )(frp, g01, g23, dts, ats, pc,
      w0, w1, w2, w3, m2, b2, cw, ct, gm, bt)


def _tc_half2(frp, g01, g23, dts, ats,
              w0, w1, w2, w3, m2, b2, gm, bt, prev, off):
    gmap = lambda i: (i, 0)
    tmap = lambda i: (i, 0, 0)
    return pl.pallas_call(
        _tc_body_ev,
        grid=(TH,),
        in_specs=[
            pl.BlockSpec(memory_space=pltpu.SMEM),
        ] + [pl.BlockSpec((B, D), gmap)] * 2 + [
            pl.BlockSpec((1, 1, B), tmap),
            pl.BlockSpec((1, 1, B), tmap),
        ] + _WSPECS + [
            _FULL((1, D)), _FULL((1, D)),
            pl.BlockSpec(memory_space=pl.ANY),
        ],
        out_specs=pl.BlockSpec((B, D), lambda i: (i + off, 0)),
        out_shape=jax.ShapeDtypeStruct(((T + 1) * B, D), jnp.float32),
        input_output_aliases={13: 0},
    )(frp, g01, g23, dts, ats,
      w0, w1, w2, w3, m2, b2, gm, bt, prev)


def _prep_params(rel_lin_w, rel_lin_b, rel_freq_w, rel_freq_b,
                 abs_lin_w, abs_lin_b, abs_freq_w, abs_freq_b,
                 time_proj_w, ctx_token, context_proj_w,
                 final_proj_w, final_proj_b, ln_gamma, ln_beta):
    wt = final_proj_w[:, 4 * D:5 * D]
    m = time_proj_w.T @ wt.T
    b2 = (final_proj_b + rel_lin_b[0] * m[0] + abs_lin_b[0] * m[8]).reshape(1, D)
    m2 = jnp.concatenate([
        (rel_lin_w[0, 0] * m[0]).reshape(1, D),
        (abs_lin_w[0, 0] * m[8]).reshape(1, D),
        m[1:8], m[9:16]], axis=0)
    frp = jnp.zeros((32,), jnp.float32)
    frp = frp.at[0:7].set(rel_freq_w[:, 0]).at[8:15].set(rel_freq_b)
    frp = frp.at[16:23].set(abs_freq_w[:, 0]).at[24:31].set(abs_freq_b)
    ws = [final_proj_w[:, k * D:(k + 1) * D].T.astype(jnp.bfloat16)
          for k in range(4)]
    cw = context_proj_w.T
    ct = ctx_token.reshape(1, D)
    gm = ln_gamma.reshape(1, D)
    bt = ln_beta.reshape(1, D)
    return frp, ws, m2, b2, cw, ct, gm, bt


def kernel(raw_concept_ids, concept_ids, value_ids, position_ids, delta_ts,
           abs_ts, patient_contexts, raw_table, con_table, val_table,
           pos_table, rel_lin_w, rel_lin_b, rel_freq_w, rel_freq_b,
           abs_lin_w, abs_lin_b, abs_freq_w, abs_freq_b, time_proj_w,
           ctx_token, context_proj_w, final_proj_w, final_proj_b,
           ln_gamma, ln_beta):
    nchw_h = NH // NW // CH
    id_in = (raw_concept_ids, concept_ids, value_ids, position_ids)
    ids = [[a[:, h * TH:(h + 1) * TH].astype(jnp.int32).T.reshape(
        NW, nchw_h, CH) for a in id_in] for h in range(HALVES)]
    tabs = (raw_table, con_table, val_table, pos_table)
    gh = [_sc_gather(NH)(*tabs, *ids[h]) for h in range(HALVES)]
    frp, ws, m2, b2, cw, ct, gm, bt = _prep_params(
        rel_lin_w, rel_lin_b, rel_freq_w, rel_freq_b,
        abs_lin_w, abs_lin_b, abs_freq_w, abs_freq_b,
        time_proj_w, ctx_token, context_proj_w,
        final_proj_w, final_proj_b, ln_gamma, ln_beta)
    dts = delta_ts.T.reshape(HALVES, TH, 1, B)
    ats = abs_ts.T.reshape(HALVES, TH, 1, B)
    out = _tc_half1(frp, gh[0][0], gh[0][1], dts[0], ats[0], patient_contexts,
                    *ws, m2, b2, cw, ct, gm, bt)
    for h in range(1, HALVES):
        out = _tc_half2(frp, gh[h][0], gh[h][1], dts[h], ats[h],
                        *ws, m2, b2, gm, bt, out, 1 + h * TH)
    return out.reshape(T + 1, B, D).transpose(1, 0, 2)
